# baseline (speedup 1.0000x reference)
import math

import jax
import jax.numpy as jnp
from jax.experimental import pallas as pl
from jax.experimental.pallas import tpu as pltpu

_COULOMB = 14.399645478425668
_SCF = _COULOMB / 1.0
_B = 16
_NPER = 256
_D = 128
_NSPEC = 4
_SQRT2 = math.sqrt(2.0)
_SQRTPI = math.sqrt(math.pi)


def _erf(x):
    t = 1.0 / (1.0 + 0.3275911 * x)
    poly = t * (0.254829592 + t * (-0.284496736 + t * (1.421413741
               + t * (-1.453152027 + t * 1.061405429))))
    return 1.0 - poly * jnp.exp(-x * x)


def _qeq_kernel(xf_ref, px_ref, py_ref, pz_ref, at_ref, tc_ref, w_ref,
                sigma_ref, hard_ref, out_ref, caug_ref, corig_ref, z_ref,
                rs_ref):
    at = at_ref[...]
    sig = jnp.zeros((_B, _NPER), jnp.float32)
    hrd = jnp.zeros((_B, _NPER), jnp.float32)
    for t in range(_NSPEC):
        sig = jnp.where(at == t, sigma_ref[t], sig)
        hrd = jnp.where(at == t, hard_ref[t], hrd)

    w = w_ref[0, :]
    chi = jnp.sum(xf_ref[...] * w[None, None, :], axis=-1)

    px = px_ref[...]
    py = py_ref[...]
    pz = pz_ref[...]
    dx = px[:, :, None] - px[:, None, :] + 1e-6
    dy = py[:, :, None] - py[:, None, :] + 1e-6
    dz = pz[:, :, None] - pz[:, None, :] + 1e-6
    d = jnp.sqrt(dx * dx + dy * dy + dz * dz)
    s2 = sig * sig
    gam = jnp.sqrt(s2[:, :, None] + s2[:, None, :])
    vals = _SCF * _erf(d / (_SQRT2 * gam)) / d

    ii = jax.lax.broadcasted_iota(jnp.int32, (_NPER, _NPER), 0)
    jj = jax.lax.broadcasted_iota(jnp.int32, (_NPER, _NPER), 1)
    diag_term = hrd * hrd + _SCF / (_SQRTPI * sig)
    c = jnp.where((ii == jj)[None, :, :], diag_term[:, :, None], vals)

    corig_ref[...] = c
    caug_ref[...] = c

    lane_c = jax.lax.broadcasted_iota(jnp.int32, (1, 1, _NPER), 2)
    lane_j = jax.lax.broadcasted_iota(jnp.int32, (1, _NPER), 1)

    z_ref[:, 0:1, :] = -chi[:, None, :]
    z_ref[:, 1:2, :] = jnp.ones((_B, 1, _NPER), jnp.float32)

    _W = 16
    for p in range(_NPER // _W):
        k0 = _W * p
        c0p = 0 if k0 < 128 else 128
        lane_w = k0 + jax.lax.broadcasted_iota(jnp.int32, (1, 1, _W), 2)

        def pf_body(t, carry, k0=k0, c0p=c0p, lane_w=lane_w):
            rowt = caug_ref[:, pl.ds(t, 1), :]
            piv = jnp.sum(jnp.where(lane_c == t, rowt, 0.0), axis=2,
                          keepdims=True)
            rowt_s = rowt * (1.0 / piv)
            rs_ref[:, pl.ds(t - k0, 1), :] = rowt_s
            fl = jnp.where(lane_w > t, rowt[:, :, k0:k0 + _W], 0.0)
            fp = jnp.swapaxes(fl, 1, 2)
            caug_ref[:, k0:k0 + _W, c0p:_NPER] = (
                caug_ref[:, k0:k0 + _W, c0p:_NPER]
                - fp * rowt_s[:, :, c0p:_NPER])
            z = z_ref[...]
            zk = jnp.sum(jnp.where(lane_c == t, z, 0.0), axis=2,
                         keepdims=True)
            z_ref[...] = z - jnp.where(lane_c > t, rowt_s * zk, 0.0)
            return carry

        jax.lax.fori_loop(k0, k0 + _W, pf_body, 0, unroll=2)

        k1 = k0 + _W
        if k1 < _NPER:
            c0t = 0 if k1 < 128 else 128
            rs = rs_ref[:, 0:_W, :]
            ftr = jnp.swapaxes(rs[:, :, k1:_NPER], 1, 2)
            u_blk = caug_ref[:, k0:k1, c0t:_NPER]
            t_upd = jax.lax.dot_general(
                ftr, u_blk, (((2,), (1,)), ((0,), (0,))),
                preferred_element_type=jnp.float32)
            caug_ref[:, k1:_NPER, c0t:_NPER] = (
                caug_ref[:, k1:_NPER, c0t:_NPER] - t_upd)
    zfull = z_ref[...]
    za = zfull[:, 0, :]
    zb = zfull[:, 1, :]

    def bwd_body(t, carry):
        xa, xb = carry
        k = _NPER - 1 - t
        rowk = caug_ref[:, pl.ds(k, 1), :]
        u = rowk[:, 0, :]
        piv = jnp.sum(jnp.where(lane_j == k, u, 0.0), axis=1,
                      keepdims=True)
        sa = jnp.sum(u * xa, axis=1, keepdims=True)
        sb = jnp.sum(u * xb, axis=1, keepdims=True)
        ra = jnp.sum(jnp.where(lane_j == k, za, 0.0), axis=1, keepdims=True)
        rb = jnp.sum(jnp.where(lane_j == k, zb, 0.0), axis=1, keepdims=True)
        xak = (ra - sa) / piv
        xbk = (rb - sb) / piv
        xa = jnp.where(lane_j == k, xak, xa)
        xb = jnp.where(lane_j == k, xbk, xb)
        return xa, xb

    xa0 = jnp.zeros((_B, _NPER), jnp.float32)
    xb0 = jnp.zeros((_B, _NPER), jnp.float32)
    xa, xb = jax.lax.fori_loop(0, _NPER, bwd_body, (xa0, xb0), unroll=False)

    q_tot = tc_ref[0, :]
    lam = (jnp.sum(xa, axis=1) - q_tot) / jnp.sum(xb, axis=1)
    q = xa - lam[:, None] * xb

    cq = jnp.sum(corig_ref[...] * q[:, None, :], axis=2)
    e = jnp.sum((0.5 * cq + chi) * q, axis=1)
    out_ref[0, :] = e


def kernel(node_features, pos, atom_types, ptr, batch, total_charge,
           W_chi, hardness, sigma):
    del ptr, batch
    xf = node_features.reshape(_B, _NPER, _D)
    px = pos[:, 0].reshape(_B, _NPER)
    py = pos[:, 1].reshape(_B, _NPER)
    pz = pos[:, 2].reshape(_B, _NPER)
    at = atom_types.reshape(_B, _NPER)
    tc = total_charge.reshape(1, _B)
    w = W_chi.reshape(1, _D)

    out = pl.pallas_call(
        _qeq_kernel,
        out_shape=jax.ShapeDtypeStruct((1, _B), jnp.float32),
        in_specs=[
            pl.BlockSpec(memory_space=pltpu.VMEM),
            pl.BlockSpec(memory_space=pltpu.VMEM),
            pl.BlockSpec(memory_space=pltpu.VMEM),
            pl.BlockSpec(memory_space=pltpu.VMEM),
            pl.BlockSpec(memory_space=pltpu.VMEM),
            pl.BlockSpec(memory_space=pltpu.VMEM),
            pl.BlockSpec(memory_space=pltpu.VMEM),
            pl.BlockSpec(memory_space=pltpu.SMEM),
            pl.BlockSpec(memory_space=pltpu.SMEM),
        ],
        out_specs=pl.BlockSpec(memory_space=pltpu.VMEM),
        scratch_shapes=[
            pltpu.VMEM((_B, _NPER, _NPER), jnp.float32),
            pltpu.VMEM((_B, _NPER, _NPER), jnp.float32),
            pltpu.VMEM((_B, 2, _NPER), jnp.float32),
            pltpu.VMEM((_B, 32, _NPER), jnp.float32),
        ],
    )(xf, px, py, pz, at, tc, w, sigma, hardness)

    return out.reshape(_B, 1)


# speedup vs baseline: 1.0288x; 1.0288x over previous
import math

import jax
import jax.numpy as jnp
from jax.experimental import pallas as pl
from jax.experimental.pallas import tpu as pltpu

_COULOMB = 14.399645478425668
_SCF = _COULOMB / 1.0
_B = 16
_NPER = 256
_D = 128
_NSPEC = 4
_SQRT2 = math.sqrt(2.0)
_SQRTPI = math.sqrt(math.pi)


def _erf(x):
    t = 1.0 / (1.0 + 0.3275911 * x)
    poly = t * (0.254829592 + t * (-0.284496736 + t * (1.421413741
               + t * (-1.453152027 + t * 1.061405429))))
    return 1.0 - poly * jnp.exp(-x * x)


def _qeq_kernel(xf_ref, px_ref, py_ref, pz_ref, at_ref, tc_ref, w_ref,
                sigma_ref, hard_ref, out_ref, caug_ref, corig_ref, z_ref,
                rs_ref):
    at = at_ref[...]
    sig = jnp.zeros((_B, _NPER), jnp.float32)
    hrd = jnp.zeros((_B, _NPER), jnp.float32)
    for t in range(_NSPEC):
        sig = jnp.where(at == t, sigma_ref[t], sig)
        hrd = jnp.where(at == t, hard_ref[t], hrd)

    w = w_ref[0, :]
    chi = jnp.sum(xf_ref[...] * w[None, None, :], axis=-1)

    px = px_ref[...]
    py = py_ref[...]
    pz = pz_ref[...]
    dx = px[:, :, None] - px[:, None, :] + 1e-6
    dy = py[:, :, None] - py[:, None, :] + 1e-6
    dz = pz[:, :, None] - pz[:, None, :] + 1e-6
    d = jnp.sqrt(dx * dx + dy * dy + dz * dz)
    s2 = sig * sig
    gam = jnp.sqrt(s2[:, :, None] + s2[:, None, :])
    vals = _SCF * _erf(d / (_SQRT2 * gam)) / d

    ii = jax.lax.broadcasted_iota(jnp.int32, (_NPER, _NPER), 0)
    jj = jax.lax.broadcasted_iota(jnp.int32, (_NPER, _NPER), 1)
    upper = jnp.where((ii < jj)[None, :, :], vals, 0.0)
    c = upper + jnp.swapaxes(upper, 1, 2)
    diag_term = hrd * hrd + _SCF / (_SQRTPI * sig)
    c = c + jnp.where((ii == jj)[None, :, :], diag_term[:, :, None], 0.0)

    corig_ref[...] = c
    caug_ref[...] = c

    lane_c = jax.lax.broadcasted_iota(jnp.int32, (1, 1, _NPER), 2)
    lane_j = jax.lax.broadcasted_iota(jnp.int32, (1, _NPER), 1)

    z_ref[:, 0:1, :] = -chi[:, None, :]
    z_ref[:, 1:2, :] = jnp.ones((_B, 1, _NPER), jnp.float32)

    _W = 16
    for p in range(_NPER // _W):
        k0 = _W * p
        c0p = 0 if k0 < 128 else 128
        lane_w = k0 + jax.lax.broadcasted_iota(jnp.int32, (1, 1, _W), 2)

        def pf_body(t, carry, k0=k0, c0p=c0p, lane_w=lane_w):
            rowt = caug_ref[:, pl.ds(t, 1), :]
            piv = jnp.sum(jnp.where(lane_c == t, rowt, 0.0), axis=2,
                          keepdims=True)
            rowt_s = rowt * (1.0 / piv)
            rs_ref[:, pl.ds(t - k0, 1), :] = rowt_s
            fl = jnp.where(lane_w > t, rowt[:, :, k0:k0 + _W], 0.0)
            fp = jnp.swapaxes(fl, 1, 2)
            caug_ref[:, k0:k0 + _W, c0p:_NPER] = (
                caug_ref[:, k0:k0 + _W, c0p:_NPER]
                - fp * rowt_s[:, :, c0p:_NPER])
            z = z_ref[...]
            zk = jnp.sum(jnp.where(lane_c == t, z, 0.0), axis=2,
                         keepdims=True)
            z_ref[...] = z - jnp.where(lane_c > t, rowt_s * zk, 0.0)
            return carry

        jax.lax.fori_loop(k0, k0 + _W, pf_body, 0, unroll=4)

        k1 = k0 + _W
        if k1 < _NPER:
            c0t = 0 if k1 < 128 else 128
            rs = rs_ref[:, 0:_W, :]
            ftr = jnp.swapaxes(rs[:, :, k1:_NPER], 1, 2)
            u_blk = caug_ref[:, k0:k1, c0t:_NPER]
            t_upd = jax.lax.dot_general(
                ftr, u_blk, (((2,), (1,)), ((0,), (0,))),
                preferred_element_type=jnp.float32)
            caug_ref[:, k1:_NPER, c0t:_NPER] = (
                caug_ref[:, k1:_NPER, c0t:_NPER] - t_upd)
    zfull = z_ref[...]
    za = zfull[:, 0, :]
    zb = zfull[:, 1, :]

    def bwd_body(t, carry):
        xa, xb = carry
        k = _NPER - 1 - t
        rowk = caug_ref[:, pl.ds(k, 1), :]
        u = rowk[:, 0, :]
        piv = jnp.sum(jnp.where(lane_j == k, u, 0.0), axis=1,
                      keepdims=True)
        sa = jnp.sum(u * xa, axis=1, keepdims=True)
        sb = jnp.sum(u * xb, axis=1, keepdims=True)
        ra = jnp.sum(jnp.where(lane_j == k, za, 0.0), axis=1, keepdims=True)
        rb = jnp.sum(jnp.where(lane_j == k, zb, 0.0), axis=1, keepdims=True)
        xak = (ra - sa) / piv
        xbk = (rb - sb) / piv
        xa = jnp.where(lane_j == k, xak, xa)
        xb = jnp.where(lane_j == k, xbk, xb)
        return xa, xb

    xa0 = jnp.zeros((_B, _NPER), jnp.float32)
    xb0 = jnp.zeros((_B, _NPER), jnp.float32)
    xa, xb = jax.lax.fori_loop(0, _NPER, bwd_body, (xa0, xb0), unroll=False)

    q_tot = tc_ref[0, :]
    lam = (jnp.sum(xa, axis=1) - q_tot) / jnp.sum(xb, axis=1)
    q = xa - lam[:, None] * xb

    cq = jnp.sum(corig_ref[...] * q[:, None, :], axis=2)
    e = jnp.sum((0.5 * cq + chi) * q, axis=1)
    out_ref[0, :] = e


def kernel(node_features, pos, atom_types, ptr, batch, total_charge,
           W_chi, hardness, sigma):
    del ptr, batch
    xf = node_features.reshape(_B, _NPER, _D)
    px = pos[:, 0].reshape(_B, _NPER)
    py = pos[:, 1].reshape(_B, _NPER)
    pz = pos[:, 2].reshape(_B, _NPER)
    at = atom_types.reshape(_B, _NPER)
    tc = total_charge.reshape(1, _B)
    w = W_chi.reshape(1, _D)

    out = pl.pallas_call(
        _qeq_kernel,
        out_shape=jax.ShapeDtypeStruct((1, _B), jnp.float32),
        in_specs=[
            pl.BlockSpec(memory_space=pltpu.VMEM),
            pl.BlockSpec(memory_space=pltpu.VMEM),
            pl.BlockSpec(memory_space=pltpu.VMEM),
            pl.BlockSpec(memory_space=pltpu.VMEM),
            pl.BlockSpec(memory_space=pltpu.VMEM),
            pl.BlockSpec(memory_space=pltpu.VMEM),
            pl.BlockSpec(memory_space=pltpu.VMEM),
            pl.BlockSpec(memory_space=pltpu.SMEM),
            pl.BlockSpec(memory_space=pltpu.SMEM),
        ],
        out_specs=pl.BlockSpec(memory_space=pltpu.VMEM),
        scratch_shapes=[
            pltpu.VMEM((_B, _NPER, _NPER), jnp.float32),
            pltpu.VMEM((_B, _NPER, _NPER), jnp.float32),
            pltpu.VMEM((_B, 2, _NPER), jnp.float32),
            pltpu.VMEM((_B, 32, _NPER), jnp.float32),
        ],
    )(xf, px, py, pz, at, tc, w, sigma, hardness)

    return out.reshape(_B, 1)


# speedup vs baseline: 1.0606x; 1.0309x over previous
import math

import jax
import jax.numpy as jnp
from jax.experimental import pallas as pl
from jax.experimental.pallas import tpu as pltpu

_COULOMB = 14.399645478425668
_SCF = _COULOMB / 1.0
_B = 16
_NPER = 256
_D = 128
_NSPEC = 4
_SQRT2 = math.sqrt(2.0)
_SQRTPI = math.sqrt(math.pi)


def _erf(x):
    t = 1.0 / (1.0 + 0.3275911 * x)
    poly = t * (0.254829592 + t * (-0.284496736 + t * (1.421413741
               + t * (-1.453152027 + t * 1.061405429))))
    return 1.0 - poly * jnp.exp(-x * x)


def _qeq_kernel(xf_ref, px_ref, py_ref, pz_ref, at_ref, tc_ref, w_ref,
                sigma_ref, hard_ref, out_ref, caug_ref, corig_ref, z_ref,
                rs_ref, x_ref):
    at = at_ref[...]
    sig = jnp.zeros((_B, _NPER), jnp.float32)
    hrd = jnp.zeros((_B, _NPER), jnp.float32)
    for t in range(_NSPEC):
        sig = jnp.where(at == t, sigma_ref[t], sig)
        hrd = jnp.where(at == t, hard_ref[t], hrd)

    w = w_ref[0, :]
    chi = jnp.sum(xf_ref[...] * w[None, None, :], axis=-1)

    px = px_ref[...]
    py = py_ref[...]
    pz = pz_ref[...]
    dx = px[:, :, None] - px[:, None, :] + 1e-6
    dy = py[:, :, None] - py[:, None, :] + 1e-6
    dz = pz[:, :, None] - pz[:, None, :] + 1e-6
    d = jnp.sqrt(dx * dx + dy * dy + dz * dz)
    s2 = sig * sig
    gam = jnp.sqrt(s2[:, :, None] + s2[:, None, :])
    vals = _SCF * _erf(d / (_SQRT2 * gam)) / d

    ii = jax.lax.broadcasted_iota(jnp.int32, (_NPER, _NPER), 0)
    jj = jax.lax.broadcasted_iota(jnp.int32, (_NPER, _NPER), 1)
    upper = jnp.where((ii < jj)[None, :, :], vals, 0.0)
    c = upper + jnp.swapaxes(upper, 1, 2)
    diag_term = hrd * hrd + _SCF / (_SQRTPI * sig)
    c = c + jnp.where((ii == jj)[None, :, :], diag_term[:, :, None], 0.0)

    corig_ref[...] = c
    caug_ref[...] = c

    lane_c = jax.lax.broadcasted_iota(jnp.int32, (1, 1, _NPER), 2)

    z_ref[:, 0:1, :] = -chi[:, None, :]
    z_ref[:, 1:2, :] = jnp.ones((_B, 1, _NPER), jnp.float32)

    _W = 16
    for p in range(_NPER // _W):
        k0 = _W * p
        c0p = 0 if k0 < 128 else 128
        lane_w = k0 + jax.lax.broadcasted_iota(jnp.int32, (1, 1, _W), 2)

        def pf_body(t, carry, k0=k0, c0p=c0p, lane_w=lane_w):
            rowt = caug_ref[:, pl.ds(t, 1), :]
            piv = jnp.sum(jnp.where(lane_c == t, rowt, 0.0), axis=2,
                          keepdims=True)
            rowt_s = rowt * (1.0 / piv)
            rs_ref[:, pl.ds(t - k0, 1), :] = rowt_s
            fl = jnp.where(lane_w > t, rowt[:, :, k0:k0 + _W], 0.0)
            fp = jnp.swapaxes(fl, 1, 2)
            caug_ref[:, k0:k0 + _W, c0p:_NPER] = (
                caug_ref[:, k0:k0 + _W, c0p:_NPER]
                - fp * rowt_s[:, :, c0p:_NPER])
            z = z_ref[...]
            zk = jnp.sum(jnp.where(lane_c == t, z, 0.0), axis=2,
                         keepdims=True)
            z_ref[...] = z - jnp.where(lane_c > t, rowt_s * zk, 0.0)
            return carry

        jax.lax.fori_loop(k0, k0 + _W, pf_body, 0, unroll=4)

        k1 = k0 + _W
        if k1 < _NPER:
            c0t = 0 if k1 < 128 else 128
            rs = rs_ref[:, 0:_W, :]
            ftr = jnp.swapaxes(rs[:, :, k1:_NPER], 1, 2)
            u_blk = caug_ref[:, k0:k1, c0t:_NPER]
            t_upd = jax.lax.dot_general(
                ftr, u_blk, (((2,), (1,)), ((0,), (0,))),
                preferred_element_type=jnp.float32)
            caug_ref[:, k1:_NPER, c0t:_NPER] = (
                caug_ref[:, k1:_NPER, c0t:_NPER] - t_upd)

    x_ref[...] = jnp.zeros((_B, 2, _NPER), jnp.float32)

    def bwd_body(t, carry):
        k = _NPER - 1 - t
        rowk = caug_ref[:, pl.ds(k, 1), :]
        piv = jnp.sum(jnp.where(lane_c == k, rowk, 0.0), axis=2,
                      keepdims=True)
        x = x_ref[...]
        s = jnp.sum(rowk * x, axis=2, keepdims=True)
        r = jnp.sum(jnp.where(lane_c == k, z_ref[...], 0.0), axis=2,
                    keepdims=True)
        xk = (r - s) / piv
        x_ref[...] = jnp.where(lane_c == k, xk, x)
        return carry

    jax.lax.fori_loop(0, _NPER, bwd_body, 0, unroll=2)
    xfull = x_ref[...]
    xa = xfull[:, 0, :]
    xb = xfull[:, 1, :]

    q_tot = tc_ref[0, :]
    lam = (jnp.sum(xa, axis=1) - q_tot) / jnp.sum(xb, axis=1)
    q = xa - lam[:, None] * xb

    cq = jnp.sum(corig_ref[...] * q[:, None, :], axis=2)
    e = jnp.sum((0.5 * cq + chi) * q, axis=1)
    out_ref[0, :] = e


def kernel(node_features, pos, atom_types, ptr, batch, total_charge,
           W_chi, hardness, sigma):
    del ptr, batch
    xf = node_features.reshape(_B, _NPER, _D)
    px = pos[:, 0].reshape(_B, _NPER)
    py = pos[:, 1].reshape(_B, _NPER)
    pz = pos[:, 2].reshape(_B, _NPER)
    at = atom_types.reshape(_B, _NPER)
    tc = total_charge.reshape(1, _B)
    w = W_chi.reshape(1, _D)

    out = pl.pallas_call(
        _qeq_kernel,
        out_shape=jax.ShapeDtypeStruct((1, _B), jnp.float32),
        in_specs=[
            pl.BlockSpec(memory_space=pltpu.VMEM),
            pl.BlockSpec(memory_space=pltpu.VMEM),
            pl.BlockSpec(memory_space=pltpu.VMEM),
            pl.BlockSpec(memory_space=pltpu.VMEM),
            pl.BlockSpec(memory_space=pltpu.VMEM),
            pl.BlockSpec(memory_space=pltpu.VMEM),
            pl.BlockSpec(memory_space=pltpu.VMEM),
            pl.BlockSpec(memory_space=pltpu.SMEM),
            pl.BlockSpec(memory_space=pltpu.SMEM),
        ],
        out_specs=pl.BlockSpec(memory_space=pltpu.VMEM),
        scratch_shapes=[
            pltpu.VMEM((_B, _NPER, _NPER), jnp.float32),
            pltpu.VMEM((_B, _NPER, _NPER), jnp.float32),
            pltpu.VMEM((_B, 2, _NPER), jnp.float32),
            pltpu.VMEM((_B, 32, _NPER), jnp.float32),
            pltpu.VMEM((_B, 2, _NPER), jnp.float32),
        ],
    )(xf, px, py, pz, at, tc, w, sigma, hardness)

    return out.reshape(_B, 1)
